# baseline (speedup 1.0000x reference)
import functools

import jax
import jax.numpy as jnp
from jax import lax
from jax.experimental import pallas as pl
from jax.experimental.pallas import tpu as pltpu
from jax.experimental.pallas import tpu_sc as plsc

_N_E = 1024
_E_DIM = 64
_B = 32
_T = 576
_N_TOK = _B * _T

_BT = 18432
_NT = _N_TOK // _BT
_RC = 64

_NW = 32
_BPW = _N_TOK // _NW


def _dist_argmin_body(z_ref, emb_ref, idx_ref, loss_ref, embt_ref, esq_ref,
                      rmin_ref, rtile_ref, *, nt):
    i = pl.program_id(0)

    @pl.when(i == 0)
    def _prep():
        emb = emb_ref[...]
        embt_ref[...] = emb.T
        esq_ref[...] = jnp.sum(emb * emb, axis=1)
        loss_ref[0, 0] = 0.0

    embt = embt_ref[...]
    esq = esq_ref[...]
    for c in range(_BT // _RC):
        z = z_ref[pl.ds(c * _RC, _RC), :]
        zsq = jnp.sum(z * z, axis=1, keepdims=True)
        mm = lax.dot_general(z, embt, (((1,), (0,)), ((), ())),
                             preferred_element_type=jnp.float32)
        run_min = None
        run_tile = None
        for t in range(_N_E // 128):
            mmt = lax.slice_in_dim(mm, t * 128, (t + 1) * 128, axis=1)
            dt = (zsq + esq[t * 128:(t + 1) * 128][None, :]) - 2.0 * mmt
            if t == 0:
                run_min = dt
                run_tile = jnp.zeros(dt.shape, jnp.float32)
            else:
                beat = dt < run_min
                run_tile = jnp.where(beat, float(t), run_tile)
                run_min = jnp.minimum(run_min, dt)
        rmin_ref[pl.ds(c * _RC, _RC), :] = run_min
        rtile_ref[pl.ds(c * _RC, _RC), :] = run_tile

    rmin = rmin_ref[...]
    rtile = rtile_ref[...]
    m = jnp.min(rmin, axis=1, keepdims=True)
    lane = lax.broadcasted_iota(jnp.int32, rmin.shape, 1).astype(jnp.float32)
    idx_lane = rtile * 128.0 + lane
    idxf = jnp.min(jnp.where(rmin == m, idx_lane, float(_N_E)), axis=1)
    idx_ref[...] = idxf.astype(jnp.int32)
    loss_ref[0, 0] += jnp.sum(m)

    @pl.when(i == nt - 1)
    def _finish():
        loss_ref[0, 0] = loss_ref[0, 0] * (1.25 / float(_N_TOK * _E_DIM))


def _dist_argmin(z_flat, embedding, interpret=False):
    nt = z_flat.shape[0] // _BT
    return pl.pallas_call(
        functools.partial(_dist_argmin_body, nt=nt),
        grid=(nt,),
        in_specs=[
            pl.BlockSpec((_BT, _E_DIM), lambda i: (i, 0)),
            pl.BlockSpec((_N_E, _E_DIM), lambda i: (0, 0)),
        ],
        out_specs=[
            pl.BlockSpec((_BT,), lambda i: (i,)),
            pl.BlockSpec(memory_space=pltpu.SMEM),
        ],
        out_shape=[
            jax.ShapeDtypeStruct((nt * _BT,), jnp.int32),
            jax.ShapeDtypeStruct((1, 1), jnp.float32),
        ],
        scratch_shapes=[
            pltpu.VMEM((_E_DIM, _N_E), jnp.float32),
            pltpu.VMEM((_N_E,), jnp.float32),
            pltpu.VMEM((_BT, 128), jnp.float32),
            pltpu.VMEM((_BT, 128), jnp.float32),
        ],
        interpret=interpret,
    )(z_flat, embedding)


@functools.cache
def _sc_gather_kernel(n_rows):
    bpw = n_rows // _NW
    mesh = plsc.VectorSubcoreMesh(core_axis_name="c", subcore_axis_name="s")

    @functools.partial(
        pl.kernel,
        out_type=jax.ShapeDtypeStruct((n_rows, _E_DIM), jnp.float32),
        mesh=mesh,
        scratch_types=[
            pltpu.VMEM((bpw,), jnp.int32),
            pltpu.VMEM((bpw, _E_DIM), jnp.float32),
            pltpu.SemaphoreType.DMA,
        ],
        compiler_params=pltpu.CompilerParams(use_tc_tiling_on_sc=False),
    )
    def _sc_gather(table_hbm, idx_hbm, out_hbm, idx_v, rows_v, sem):
        wid = lax.axis_index("s") * 2 + lax.axis_index("c")
        base = wid * bpw
        pltpu.sync_copy(idx_hbm.at[pl.ds(base, bpw)], idx_v)
        pltpu.async_copy(table_hbm.at[idx_v], rows_v, sem).wait()
        pltpu.sync_copy(rows_v, out_hbm.at[pl.ds(base, bpw)])

    return _sc_gather


def kernel(z, embedding):
    z_flat = z.reshape(_N_TOK, _E_DIM)
    idx, loss = _dist_argmin(z_flat, embedding)
    z_q = _sc_gather_one_core(_N_TOK)(embedding, idx)
    return z_q.reshape(z.shape), idx, loss[0, 0]


@functools.cache
def _sc_gather_one_core(n_rows):
    bpw = n_rows // 16
    mesh = plsc.VectorSubcoreMesh(
        core_axis_name="c", subcore_axis_name="s", num_cores=1
    )

    @functools.partial(
        pl.kernel,
        out_type=jax.ShapeDtypeStruct((n_rows, _E_DIM), jnp.float32),
        mesh=mesh,
        scratch_types=[
            pltpu.VMEM((bpw,), jnp.int32),
            pltpu.VMEM((bpw, _E_DIM), jnp.float32),
            pltpu.SemaphoreType.DMA,
        ],
        compiler_params=pltpu.CompilerParams(use_tc_tiling_on_sc=False),
    )
    def _sc_gather1(table_hbm, idx_hbm, out_hbm, idx_v, rows_v, sem):
        base = lax.axis_index("s") * bpw
        pltpu.sync_copy(idx_hbm.at[pl.ds(base, bpw)], idx_v)
        pltpu.async_copy(table_hbm.at[idx_v], rows_v, sem).wait()
        pltpu.sync_copy(rows_v, out_hbm.at[pl.ds(base, bpw)])

    return _sc_gather1


# speedup vs baseline: 1.0613x; 1.0613x over previous
import functools

import jax
import jax.numpy as jnp
from jax import lax
from jax.experimental import pallas as pl
from jax.experimental.pallas import tpu as pltpu
from jax.experimental.pallas import tpu_sc as plsc

_N_E = 1024
_E_DIM = 64
_B = 32
_T = 576
_N_TOK = _B * _T

_BT = 9216
_NT = _N_TOK // _BT
_RC = 128

_NW = 32
_BPW = _N_TOK // _NW


def _dist_argmin_body(z_ref, emb_ref, idx_ref, loss_ref, embt_ref, esq_ref,
                      rmin_ref, rtile_ref, *, nt):
    i = pl.program_id(0)

    @pl.when(i == 0)
    def _prep():
        emb = emb_ref[...]
        embt_ref[...] = emb.T
        esq_ref[...] = jnp.sum(emb * emb, axis=1)
        loss_ref[0, 0] = 0.0

    embt = embt_ref[...]
    esq = esq_ref[...]
    for c in range(_BT // _RC):
        z = z_ref[pl.ds(c * _RC, _RC), :]
        zsq = jnp.sum(z * z, axis=1, keepdims=True)
        mm = lax.dot_general(z, embt, (((1,), (0,)), ((), ())),
                             preferred_element_type=jnp.float32)
        run_min = None
        run_tile = None
        for t in range(_N_E // 128):
            mmt = lax.slice_in_dim(mm, t * 128, (t + 1) * 128, axis=1)
            dt = (zsq + esq[t * 128:(t + 1) * 128][None, :]) - 2.0 * mmt
            if t == 0:
                run_min = dt
                run_tile = jnp.zeros(dt.shape, jnp.float32)
            else:
                beat = dt < run_min
                run_tile = jnp.where(beat, float(t), run_tile)
                run_min = jnp.minimum(run_min, dt)
        rmin_ref[pl.ds(c * _RC, _RC), :] = run_min
        rtile_ref[pl.ds(c * _RC, _RC), :] = run_tile

    rmin = rmin_ref[...]
    rtile = rtile_ref[...]
    m = jnp.min(rmin, axis=1, keepdims=True)
    lane = lax.broadcasted_iota(jnp.int32, rmin.shape, 1).astype(jnp.float32)
    idx_lane = rtile * 128.0 + lane
    idxf = jnp.min(jnp.where(rmin == m, idx_lane, float(_N_E)), axis=1)
    idx_ref[...] = idxf.astype(jnp.int32)
    loss_ref[0, 0] += jnp.sum(m)

    @pl.when(i == nt - 1)
    def _finish():
        loss_ref[0, 0] = loss_ref[0, 0] * (1.25 / float(_N_TOK * _E_DIM))


def _dist_argmin(z_flat, embedding, interpret=False):
    nt = z_flat.shape[0] // _BT
    return pl.pallas_call(
        functools.partial(_dist_argmin_body, nt=nt),
        grid=(nt,),
        in_specs=[
            pl.BlockSpec((_BT, _E_DIM), lambda i: (i, 0)),
            pl.BlockSpec((_N_E, _E_DIM), lambda i: (0, 0)),
        ],
        out_specs=[
            pl.BlockSpec((_BT,), lambda i: (i,)),
            pl.BlockSpec(memory_space=pltpu.SMEM),
        ],
        out_shape=[
            jax.ShapeDtypeStruct((nt * _BT,), jnp.int32),
            jax.ShapeDtypeStruct((1, 1), jnp.float32),
        ],
        scratch_shapes=[
            pltpu.VMEM((_E_DIM, _N_E), jnp.float32),
            pltpu.VMEM((_N_E,), jnp.float32),
            pltpu.VMEM((_BT, 128), jnp.float32),
            pltpu.VMEM((_BT, 128), jnp.float32),
        ],
        interpret=interpret,
    )(z_flat, embedding)


@functools.cache
def _sc_gather_kernel(n_rows):
    bpw = n_rows // _NW
    mesh = plsc.VectorSubcoreMesh(core_axis_name="c", subcore_axis_name="s")

    @functools.partial(
        pl.kernel,
        out_type=jax.ShapeDtypeStruct((n_rows, _E_DIM), jnp.float32),
        mesh=mesh,
        scratch_types=[
            pltpu.VMEM((bpw,), jnp.int32),
            pltpu.VMEM((bpw, _E_DIM), jnp.float32),
            pltpu.SemaphoreType.DMA,
        ],
        compiler_params=pltpu.CompilerParams(use_tc_tiling_on_sc=False),
    )
    def _sc_gather(table_hbm, idx_hbm, out_hbm, idx_v, rows_v, sem):
        wid = lax.axis_index("s") * 2 + lax.axis_index("c")
        base = wid * bpw
        pltpu.sync_copy(idx_hbm.at[pl.ds(base, bpw)], idx_v)
        pltpu.async_copy(table_hbm.at[idx_v], rows_v, sem).wait()
        pltpu.sync_copy(rows_v, out_hbm.at[pl.ds(base, bpw)])

    return _sc_gather


def kernel(z, embedding):
    z_flat = z.reshape(_N_TOK, _E_DIM)
    idx, loss = _dist_argmin(z_flat, embedding)
    z_q = _sc_gather_one_core(_N_TOK)(embedding, idx)
    return z_q.reshape(z.shape), idx, loss[0, 0]


@functools.cache
def _sc_gather_one_core(n_rows):
    bpw = n_rows // 16
    mesh = plsc.VectorSubcoreMesh(
        core_axis_name="c", subcore_axis_name="s", num_cores=1
    )

    @functools.partial(
        pl.kernel,
        out_type=jax.ShapeDtypeStruct((n_rows, _E_DIM), jnp.float32),
        mesh=mesh,
        scratch_types=[
            pltpu.VMEM((bpw,), jnp.int32),
            pltpu.VMEM((bpw, _E_DIM), jnp.float32),
            pltpu.SemaphoreType.DMA,
        ],
        compiler_params=pltpu.CompilerParams(use_tc_tiling_on_sc=False),
    )
    def _sc_gather1(table_hbm, idx_hbm, out_hbm, idx_v, rows_v, sem):
        base = lax.axis_index("s") * bpw
        pltpu.sync_copy(idx_hbm.at[pl.ds(base, bpw)], idx_v)
        pltpu.async_copy(table_hbm.at[idx_v], rows_v, sem).wait()
        pltpu.sync_copy(rows_v, out_hbm.at[pl.ds(base, bpw)])

    return _sc_gather1
